# baseline (speedup 1.0000x reference)
import functools
import math

import jax
import jax.numpy as jnp
from jax import lax
from jax.experimental import pallas as pl
from jax.experimental.pallas import tpu as pltpu
from jax.experimental.pallas import tpu_sc as plsc

B = 32
J3 = 34
T = 256
N_STATE = 32
DEPTH = 3
ROW = J3 * T


def _ln(x, g, b):
    mu = x.mean(-1, keepdims=True)
    var = ((x - mu) ** 2).mean(-1, keepdims=True)
    return (x - mu) / jnp.sqrt(var + 1e-5) * g + b


def _silu(x):
    return x * jax.nn.sigmoid(x)


def _layer_body(x_ref, *refs, L, D, r, Cn, Q, final_ln):
    N = N_STATE
    Lp = Cn * Q
    (wixT1, wizT1, cw01, cw11, cb1, wxdT1, wxbT1, wxcT1, wdtT1, bdt1, anT1, dv1, woT1,
     wixT2, wizT2, cw02, cw12, cb2, wxdT2, wxbT2, wxcT2, wdtT2, bdt2, anT2, dv2, woT2,
     ln1g, ln1b, ffw1T, ffb1, ffw2T, ffb2, ln2g, ln2b, fg, fb,
     o_ref, dt_s, dtxc_s, bm_s, cm_s) = refs

    LB = Lp * B
    pads = [divmod(t, Q) for t in range(L, Lp)]
    x = x_ref[:]
    x2 = x.reshape(LB, D)

    def run_dir(wixT, wizT, cw0, cw1, cb, wxdT, wxbT, wxcT, wdtT, bdt, anT, dv,
                woT, reverse):
        xp = jnp.dot(x2, wixT[:], preferred_element_type=jnp.float32)
        z = jnp.dot(x2, wizT[:], preferred_element_type=jnp.float32)
        xp3 = xp.reshape(Lp, B, D)
        zero = jnp.zeros((1, B, D), jnp.float32)
        if not reverse:
            first = jnp.concatenate(
                [zero, xp3[(Q - 1) * Cn:(Q - 1) * Cn + Cn - 1]], axis=0)
            xsh = jnp.concatenate([first, xp3[:(Q - 1) * Cn]], axis=0)
        else:
            last = jnp.concatenate([xp3[1:Cn], zero], axis=0)
            xsh = jnp.concatenate([xp3[Cn:], last], axis=0)
        xc = xsh * cw0[:] + xp3 * cw1[:] + cb[:]
        xc = _silu(xc)
        xc2 = xc.reshape(LB, D)
        dtl = jnp.dot(xc2, wxdT[:], preferred_element_type=jnp.float32)
        bm = jnp.dot(xc2, wxbT[:], preferred_element_type=jnp.float32)
        cm = jnp.dot(xc2, wxcT[:], preferred_element_type=jnp.float32)
        dtf = jax.nn.softplus(
            jnp.dot(dtl, wdtT[:], preferred_element_type=jnp.float32) + bdt[:])
        dt_s[:] = dtf.reshape(Q, Cn, B, D)
        dtxc_s[:] = (dtf * xc2).reshape(Q, Cn, B, D)
        bm_s[:] = bm.reshape(Q, Cn, B, N)
        cm_s[:] = cm.reshape(Q, Cn, B, N)
        for (k, j) in pads:
            dt_s[j, k] = jnp.zeros((B, D), jnp.float32)
            dtxc_s[j, k] = jnp.zeros((B, D), jnp.float32)
        anT_v = anT[:]

        def dA_of(dtq):
            return jnp.exp(dtq[:, :, None, :] * anT_v[None, None, :, :])

        def dBx_of(dtxcq, bq):
            return dtxcq[:, :, None, :] * bq[:, :, :, None]

        def jj_of(i):
            return Q - 1 - i if reverse else i

        if Cn == 1:
            def step(i, h):
                jj = jj_of(i)
                dtq = dt_s[jj]
                h = dA_of(dtq) * h + dBx_of(dtxc_s[jj], bm_s[jj])
                dt_s[jj] = jnp.sum(h * cm_s[jj][:, :, :, None], axis=2)
                return h
            lax.fori_loop(0, Q, step,
                          jnp.zeros((Cn, B, N, D), jnp.float32))
        else:
            def step_a(i, carry):
                h, sdt = carry
                jj = jj_of(i)
                dtq = dt_s[jj]
                h = dA_of(dtq) * h + dBx_of(dtxc_s[jj], bm_s[jj])
                return h, sdt + dtq
            S, sdt = lax.fori_loop(
                0, Q, step_a,
                (jnp.zeros((Cn, B, N, D), jnp.float32),
                 jnp.zeros((Cn, B, D), jnp.float32)))
            P = jnp.exp(sdt[:, :, None, :] * anT_v[None, None, :, :])
            g = jnp.zeros((B, N, D), jnp.float32)
            gs = [None] * Cn
            order = range(Cn - 1, -1, -1) if reverse else range(Cn)
            for k in order:
                gs[k] = g
                g = P[k] * g + S[k]
            G = jnp.stack(gs)

            def step_c(i, h):
                jj = jj_of(i)
                dtq = dt_s[jj]
                h = dA_of(dtq) * h + dBx_of(dtxc_s[jj], bm_s[jj])
                dt_s[jj] = jnp.sum(h * cm_s[jj][:, :, :, None], axis=2)
                return h
            lax.fori_loop(0, Q, step_c, G)

        y = dt_s[:].reshape(Lp, B, D) + xc * dv[:]
        y = y * _silu(z.reshape(Lp, B, D))
        return jnp.dot(y.reshape(LB, D), woT[:], preferred_element_type=jnp.float32)

    o1 = run_dir(wixT1, wizT1, cw01, cw11, cb1, wxdT1, wxbT1, wxcT1, wdtT1,
                 bdt1, anT1, dv1, woT1, reverse=False)
    o2 = run_dir(wixT2, wizT2, cw02, cw12, cb2, wxdT2, wxbT2, wxcT2, wdtT2,
                 bdt2, anT2, dv2, woT2, reverse=True)

    xr = x + (o1 + o2).reshape(Lp, B, D)
    x1 = _ln(xr, ln1g[:], ln1b[:])
    ffh = jax.nn.gelu(
        jnp.dot(x1.reshape(LB, D), ffw1T[:], preferred_element_type=jnp.float32)
        + ffb1[:])
    ffo = jnp.dot(ffh, ffw2T[:], preferred_element_type=jnp.float32) + ffb2[:]
    out = _ln(x1 + ffo.reshape(Lp, B, D), ln2g[:], ln2b[:])
    if final_ln:
        out = _ln(out, fg[:], fb[:])
    o_ref[:] = out
    for (k, j) in pads:
        o_ref[j * Cn + k] = jnp.zeros((B, D), jnp.float32)


def _prep_mamba(p):
    r = p['W_dt'].shape[1]
    d = p['W_in'].shape[1]
    wiT = p['W_in'].T
    wxT = p['W_x'].T
    return (
        wiT[:, :d], wiT[:, d:],
        p['conv_w'][:, 0], p['conv_w'][:, 1], p['conv_b'],
        wxT[:, :r], wxT[:, r:r + N_STATE], wxT[:, r + N_STATE:],
        p['W_dt'].T, p['b_dt'],
        (-jnp.exp(p['A_log'])).T,
        p['D'], p['W_out'].T,
    )


def _layer_call(x_lbd, lp, fin_g, fin_b, L, D, r, Cn, Q, final_ln,
                interpret=False):
    N = N_STATE
    Lp = Cn * Q
    args = (x_lbd, *_prep_mamba(lp['m1']), *_prep_mamba(lp['m2']),
            lp['ln1_g'], lp['ln1_b'], lp['ff_w1'].T, lp['ff_b1'],
            lp['ff_w2'].T, lp['ff_b2'], lp['ln2_g'], lp['ln2_b'], fin_g, fin_b)
    return pl.pallas_call(
        functools.partial(_layer_body, L=L, D=D, r=r, Cn=Cn, Q=Q,
                          final_ln=final_ln),
        out_shape=jax.ShapeDtypeStruct((Lp, B, D), jnp.float32),
        scratch_shapes=[
            pltpu.VMEM((Q, Cn, B, D), jnp.float32),
            pltpu.VMEM((Q, Cn, B, D), jnp.float32),
            pltpu.VMEM((Q, Cn, B, N), jnp.float32),
            pltpu.VMEM((Q, Cn, B, N), jnp.float32),
        ],
        interpret=interpret,
    )(*args)


def _interleave(x_nat, Cn, Q):
    L = x_nat.shape[0]
    Lp = Cn * Q
    if Lp > L:
        x_nat = jnp.concatenate(
            [x_nat, jnp.zeros((Lp - L,) + x_nat.shape[1:], x_nat.dtype)], axis=0)
    return x_nat.reshape(Cn, Q, *x_nat.shape[1:]).swapaxes(0, 1).reshape(
        Lp, *x_nat.shape[1:])


def _deinterleave(y, Cn, Q, L):
    y = y.reshape(Q, Cn, *y.shape[1:]).swapaxes(0, 1).reshape(
        Cn * Q, *y.shape[1:])
    return y[:L]


def _encoder(x_lbd, ep, L, D, r, Cn, Q, interpret=False):
    h = _interleave(x_lbd, Cn, Q)
    for i, lp in enumerate(ep['layers']):
        h = _layer_call(h, lp, ep['ln_g'], ep['ln_b'], L, D, r, Cn, Q,
                        final_ln=(i == DEPTH - 1), interpret=interpret)
    return _deinterleave(h, Cn, Q, L)


def _gate_body(x_ref, gwT_ref, gb_ref, i1_ref, i2_ref):
    xv = x_ref[:]
    pooled = jnp.mean(xv, axis=1)
    logits = jnp.dot(pooled, gwT_ref[:], preferred_element_type=jnp.float32)
    logits = logits + gb_ref[:]
    mx = jnp.max(logits, axis=-1, keepdims=True)
    i4 = lax.broadcasted_iota(jnp.int32, logits.shape, 1)
    e = jnp.min(jnp.where(logits >= mx, i4, 4), axis=-1, keepdims=True)
    first_T = ((e == 0) | (e == 3)).astype(jnp.int32)
    second_S = ((e == 0) | (e == 2)).astype(jnp.int32)
    biota = lax.broadcasted_iota(jnp.int32, (B, 1), 0)
    i1_ref[:] = biota + B * (1 - first_T)
    i2_ref[:] = biota + B * (1 - second_S)


def _gate(x, gw, gb, interpret=False):
    i1, i2 = pl.pallas_call(
        _gate_body,
        out_shape=(jax.ShapeDtypeStruct((B, 1), jnp.int32),
                   jax.ShapeDtypeStruct((B, 1), jnp.int32)),
        interpret=interpret,
    )(x, gw.T, gb)
    return i1.reshape(B), i2.reshape(B)


_B_PER_W = 8
_N_SEL_W = B // _B_PER_W


def _make_select():
    mesh = plsc.VectorSubcoreMesh(core_axis_name="c", subcore_axis_name="s")

    @functools.partial(
        pl.kernel, mesh=mesh,
        out_type=jax.ShapeDtypeStruct((B, ROW), jnp.float32),
        scratch_types=[
            pltpu.VMEM((_B_PER_W,), jnp.int32),
            pltpu.VMEM((_B_PER_W, ROW), jnp.float32),
            pltpu.SemaphoreType.DMA,
        ],
    )
    def sel(table_hbm, idx_hbm, out_hbm, idx_v, rows_v, sem):
        wid = lax.axis_index("s") * 2 + lax.axis_index("c")

        @pl.when(wid < _N_SEL_W)
        def _():
            base = wid * _B_PER_W
            pltpu.sync_copy(idx_hbm.at[pl.ds(base, _B_PER_W)], idx_v)
            pltpu.async_copy(table_hbm.at[idx_v], rows_v, sem).wait()
            pltpu.sync_copy(rows_v, out_hbm.at[pl.ds(base, _B_PER_W)])

    return sel


_SEL_CACHE = []


def _select(table, idx):
    if not _SEL_CACHE:
        _SEL_CACHE.append(_make_select())
    return _SEL_CACHE[0](table, idx)


def kernel(x, params):
    pT = params['enc_T']
    pS = params['enc_S']
    rT = pT['layers'][0]['m1']['W_dt'].shape[1]
    rS = pS['layers'][0]['m1']['W_dt'].shape[1]

    encT = functools.partial(_encoder, ep=pT, L=T, D=J3, r=rT, Cn=16, Q=16)
    encS = functools.partial(_encoder, ep=pS, L=J3, D=T, r=rS, Cn=6, Q=6)

    u = encT(x.transpose(2, 0, 1))
    v = encS(x.transpose(1, 0, 2))

    i1, i2 = _gate(x, params['gate_w'], params['gate_b'])

    u_rows = u.transpose(1, 2, 0).reshape(B, ROW)
    v_rows = v.transpose(1, 0, 2).reshape(B, ROW)
    y1 = _select(jnp.concatenate([u_rows, v_rows], axis=0), i1)
    y1 = y1.reshape(B, J3, T)

    a = encS(y1.transpose(1, 0, 2))
    c = encT(y1.transpose(2, 0, 1))
    a_rows = a.transpose(1, 0, 2).reshape(B, ROW)
    c_rows = c.transpose(1, 2, 0).reshape(B, ROW)
    out = _select(jnp.concatenate([a_rows, c_rows], axis=0), i2)
    return out.reshape(B, J3, T)


# speedup vs baseline: 1.4497x; 1.4497x over previous
import functools
import math

import jax
import jax.numpy as jnp
from jax import lax
from jax.experimental import pallas as pl
from jax.experimental.pallas import tpu as pltpu
from jax.experimental.pallas import tpu_sc as plsc

B = 32
J3 = 34
T = 256
N_STATE = 32
DEPTH = 3
ROW = J3 * T


def _ln(x, g, b):
    mu = x.mean(-1, keepdims=True)
    var = ((x - mu) ** 2).mean(-1, keepdims=True)
    return (x - mu) / jnp.sqrt(var + 1e-5) * g + b


def _silu(x):
    return x * jax.nn.sigmoid(x)


def _layer_body(x_ref, *refs, L, D, r, Cn, Q, final_ln):
    N = N_STATE
    Lp = Cn * Q
    (wixT1, wizT1, cw01, cw11, cb1, wxdT1, wxbT1, wxcT1, wdtT1, bdt1, anT1, dv1, woT1,
     wixT2, wizT2, cw02, cw12, cb2, wxdT2, wxbT2, wxcT2, wdtT2, bdt2, anT2, dv2, woT2,
     ln1g, ln1b, ffw1T, ffb1, ffw2T, ffb2, ln2g, ln2b, fg, fb,
     o_ref, dt_s, dtxc_s, bm_s, cm_s) = refs

    LB = Lp * B
    pads = [divmod(t, Q) for t in range(L, Lp)]
    x = x_ref[:]
    x2 = x.reshape(LB, D)

    def run_dir(wixT, wizT, cw0, cw1, cb, wxdT, wxbT, wxcT, wdtT, bdt, anT, dv,
                woT, reverse):
        xp = jnp.dot(x2, wixT[:], preferred_element_type=jnp.float32)
        z = jnp.dot(x2, wizT[:], preferred_element_type=jnp.float32)
        xp3 = xp.reshape(Lp, B, D)
        zero = jnp.zeros((1, B, D), jnp.float32)
        if Cn == 1:
            if not reverse:
                xsh = jnp.concatenate([zero, xp3[:-1]], axis=0)
            else:
                xsh = jnp.concatenate([xp3[1:], zero], axis=0)
        elif not reverse:
            first = jnp.concatenate(
                [zero, xp3[(Q - 1) * Cn:(Q - 1) * Cn + Cn - 1]], axis=0)
            xsh = jnp.concatenate([first, xp3[:(Q - 1) * Cn]], axis=0)
        else:
            last = jnp.concatenate([xp3[1:Cn], zero], axis=0)
            xsh = jnp.concatenate([xp3[Cn:], last], axis=0)
        xc = xsh * cw0[:] + xp3 * cw1[:] + cb[:]
        xc = _silu(xc)
        xc2 = xc.reshape(LB, D)
        dtl = jnp.dot(xc2, wxdT[:], preferred_element_type=jnp.float32)
        bm = jnp.dot(xc2, wxbT[:], preferred_element_type=jnp.float32)
        cm = jnp.dot(xc2, wxcT[:], preferred_element_type=jnp.float32)
        dtf = jax.nn.softplus(
            jnp.dot(dtl, wdtT[:], preferred_element_type=jnp.float32) + bdt[:])
        dt_s[:] = dtf.reshape(Q, Cn, B, D)
        dtxc_s[:] = (dtf * xc2).reshape(Q, Cn, B, D)
        bm_s[:] = bm.reshape(Q, Cn, B, N)
        cm_s[:] = cm.reshape(Q, Cn, B, N)
        for (k, j) in pads:
            dt_s[j, k] = jnp.zeros((B, D), jnp.float32)
            dtxc_s[j, k] = jnp.zeros((B, D), jnp.float32)
        anT_v = anT[:]

        def dA_of(dtq):
            return jnp.exp(dtq[:, :, None, :] * anT_v[None, None, :, :])

        def dBx_of(dtxcq, bq):
            return dtxcq[:, :, None, :] * bq[:, :, :, None]

        def jj_of(i):
            return Q - 1 - i if reverse else i

        if Cn == 1:
            def step(i, h):
                jj = jj_of(i)
                dtq = dt_s[jj]
                h = dA_of(dtq) * h + dBx_of(dtxc_s[jj], bm_s[jj])
                dt_s[jj] = jnp.sum(h * cm_s[jj][:, :, :, None], axis=2)
                return h
            lax.fori_loop(0, Q, step,
                          jnp.zeros((Cn, B, N, D), jnp.float32))
        else:
            def step_a(i, carry):
                h, sdt = carry
                jj = jj_of(i)
                dtq = dt_s[jj]
                h = dA_of(dtq) * h + dBx_of(dtxc_s[jj], bm_s[jj])
                return h, sdt + dtq
            S, sdt = lax.fori_loop(
                0, Q, step_a,
                (jnp.zeros((Cn, B, N, D), jnp.float32),
                 jnp.zeros((Cn, B, D), jnp.float32)))
            P = jnp.exp(sdt[:, :, None, :] * anT_v[None, None, :, :])
            g = jnp.zeros((B, N, D), jnp.float32)
            gs = [None] * Cn
            order = range(Cn - 1, -1, -1) if reverse else range(Cn)
            for k in order:
                gs[k] = g
                g = P[k] * g + S[k]
            G = jnp.stack(gs)

            def step_c(i, h):
                jj = jj_of(i)
                dtq = dt_s[jj]
                h = dA_of(dtq) * h + dBx_of(dtxc_s[jj], bm_s[jj])
                dt_s[jj] = jnp.sum(h * cm_s[jj][:, :, :, None], axis=2)
                return h
            lax.fori_loop(0, Q, step_c, G)

        y = dt_s[:].reshape(Lp, B, D) + xc * dv[:]
        y = y * _silu(z.reshape(Lp, B, D))
        return jnp.dot(y.reshape(LB, D), woT[:], preferred_element_type=jnp.float32)

    o1 = run_dir(wixT1, wizT1, cw01, cw11, cb1, wxdT1, wxbT1, wxcT1, wdtT1,
                 bdt1, anT1, dv1, woT1, reverse=False)
    o2 = run_dir(wixT2, wizT2, cw02, cw12, cb2, wxdT2, wxbT2, wxcT2, wdtT2,
                 bdt2, anT2, dv2, woT2, reverse=True)

    xr = x + (o1 + o2).reshape(Lp, B, D)
    x1 = _ln(xr, ln1g[:], ln1b[:])
    ffh = jax.nn.gelu(
        jnp.dot(x1.reshape(LB, D), ffw1T[:], preferred_element_type=jnp.float32)
        + ffb1[:])
    ffo = jnp.dot(ffh, ffw2T[:], preferred_element_type=jnp.float32) + ffb2[:]
    out = _ln(x1 + ffo.reshape(Lp, B, D), ln2g[:], ln2b[:])
    if final_ln:
        out = _ln(out, fg[:], fb[:])
    o_ref[:] = out
    for (k, j) in pads:
        o_ref[j * Cn + k] = jnp.zeros((B, D), jnp.float32)


def _prep_mamba(p):
    r = p['W_dt'].shape[1]
    d = p['W_in'].shape[1]
    wiT = p['W_in'].T
    wxT = p['W_x'].T
    return (
        wiT[:, :d], wiT[:, d:],
        p['conv_w'][:, 0], p['conv_w'][:, 1], p['conv_b'],
        wxT[:, :r], wxT[:, r:r + N_STATE], wxT[:, r + N_STATE:],
        p['W_dt'].T, p['b_dt'],
        (-jnp.exp(p['A_log'])).T,
        p['D'], p['W_out'].T,
    )


def _layer_call(x_lbd, lp, fin_g, fin_b, L, D, r, Cn, Q, final_ln,
                interpret=False):
    N = N_STATE
    Lp = Cn * Q
    args = (x_lbd, *_prep_mamba(lp['m1']), *_prep_mamba(lp['m2']),
            lp['ln1_g'], lp['ln1_b'], lp['ff_w1'].T, lp['ff_b1'],
            lp['ff_w2'].T, lp['ff_b2'], lp['ln2_g'], lp['ln2_b'], fin_g, fin_b)
    return pl.pallas_call(
        functools.partial(_layer_body, L=L, D=D, r=r, Cn=Cn, Q=Q,
                          final_ln=final_ln),
        out_shape=jax.ShapeDtypeStruct((Lp, B, D), jnp.float32),
        scratch_shapes=[
            pltpu.VMEM((Q, Cn, B, D), jnp.float32),
            pltpu.VMEM((Q, Cn, B, D), jnp.float32),
            pltpu.VMEM((Q, Cn, B, N), jnp.float32),
            pltpu.VMEM((Q, Cn, B, N), jnp.float32),
        ],
        interpret=interpret,
    )(*args)


def _interleave(x_nat, Cn, Q):
    L = x_nat.shape[0]
    Lp = Cn * Q
    if Lp > L:
        x_nat = jnp.concatenate(
            [x_nat, jnp.zeros((Lp - L,) + x_nat.shape[1:], x_nat.dtype)], axis=0)
    return x_nat.reshape(Cn, Q, *x_nat.shape[1:]).swapaxes(0, 1).reshape(
        Lp, *x_nat.shape[1:])


def _deinterleave(y, Cn, Q, L):
    y = y.reshape(Q, Cn, *y.shape[1:]).swapaxes(0, 1).reshape(
        Cn * Q, *y.shape[1:])
    return y[:L]


def _encoder(x_lbd, ep, L, D, r, Cn, Q, interpret=False):
    h = _interleave(x_lbd, Cn, Q)
    for i, lp in enumerate(ep['layers']):
        h = _layer_call(h, lp, ep['ln_g'], ep['ln_b'], L, D, r, Cn, Q,
                        final_ln=(i == DEPTH - 1), interpret=interpret)
    return _deinterleave(h, Cn, Q, L)


def _gate_body(x_ref, gwT_ref, gb_ref, i1_ref, i2_ref):
    xv = x_ref[:]
    pooled = jnp.mean(xv, axis=1)
    logits = jnp.dot(pooled, gwT_ref[:], preferred_element_type=jnp.float32)
    logits = logits + gb_ref[:]
    mx = jnp.max(logits, axis=-1, keepdims=True)
    i4 = lax.broadcasted_iota(jnp.int32, logits.shape, 1)
    e = jnp.min(jnp.where(logits >= mx, i4, 4), axis=-1, keepdims=True)
    first_T = ((e == 0) | (e == 3)).astype(jnp.int32)
    second_S = ((e == 0) | (e == 2)).astype(jnp.int32)
    biota = lax.broadcasted_iota(jnp.int32, (B, 1), 0)
    i1_ref[:] = biota + B * (1 - first_T)
    i2_ref[:] = biota + B * (1 - second_S)


def _gate(x, gw, gb, interpret=False):
    i1, i2 = pl.pallas_call(
        _gate_body,
        out_shape=(jax.ShapeDtypeStruct((B, 1), jnp.int32),
                   jax.ShapeDtypeStruct((B, 1), jnp.int32)),
        interpret=interpret,
    )(x, gw.T, gb)
    return i1.reshape(B), i2.reshape(B)


_B_PER_W = 8
_N_SEL_W = B // _B_PER_W


def _make_select():
    mesh = plsc.VectorSubcoreMesh(core_axis_name="c", subcore_axis_name="s")

    @functools.partial(
        pl.kernel, mesh=mesh,
        out_type=jax.ShapeDtypeStruct((B, ROW), jnp.float32),
        scratch_types=[
            pltpu.VMEM((_B_PER_W,), jnp.int32),
            pltpu.VMEM((_B_PER_W, ROW), jnp.float32),
            pltpu.SemaphoreType.DMA,
        ],
    )
    def sel(table_hbm, idx_hbm, out_hbm, idx_v, rows_v, sem):
        wid = lax.axis_index("s") * 2 + lax.axis_index("c")

        @pl.when(wid < _N_SEL_W)
        def _():
            base = wid * _B_PER_W
            pltpu.sync_copy(idx_hbm.at[pl.ds(base, _B_PER_W)], idx_v)
            pltpu.async_copy(table_hbm.at[idx_v], rows_v, sem).wait()
            pltpu.sync_copy(rows_v, out_hbm.at[pl.ds(base, _B_PER_W)])

    return sel


_SEL_CACHE = []


def _select(table, idx):
    if not _SEL_CACHE:
        _SEL_CACHE.append(_make_select())
    return _SEL_CACHE[0](table, idx)


def kernel(x, params):
    pT = params['enc_T']
    pS = params['enc_S']
    rT = pT['layers'][0]['m1']['W_dt'].shape[1]
    rS = pS['layers'][0]['m1']['W_dt'].shape[1]

    encT = functools.partial(_encoder, ep=pT, L=T, D=J3, r=rT, Cn=1, Q=T)
    encS = functools.partial(_encoder, ep=pS, L=J3, D=T, r=rS, Cn=1, Q=J3)

    u = encT(x.transpose(2, 0, 1))
    v = encS(x.transpose(1, 0, 2))

    i1, i2 = _gate(x, params['gate_w'], params['gate_b'])

    u_rows = u.transpose(1, 2, 0).reshape(B, ROW)
    v_rows = v.transpose(1, 0, 2).reshape(B, ROW)
    y1 = _select(jnp.concatenate([u_rows, v_rows], axis=0), i1)
    y1 = y1.reshape(B, J3, T)

    a = encS(y1.transpose(1, 0, 2))
    c = encT(y1.transpose(2, 0, 1))
    a_rows = a.transpose(1, 0, 2).reshape(B, ROW)
    c_rows = c.transpose(1, 2, 0).reshape(B, ROW)
    out = _select(jnp.concatenate([a_rows, c_rows], axis=0), i2)
    return out.reshape(B, J3, T)


# speedup vs baseline: 1.7093x; 1.1791x over previous
import functools
import math

import jax
import jax.numpy as jnp
from jax import lax
from jax.experimental import pallas as pl
from jax.experimental.pallas import tpu as pltpu
from jax.experimental.pallas import tpu_sc as plsc

B = 32
J3 = 34
T = 256
N_STATE = 32
DEPTH = 3
ROW = J3 * T


def _ln(x, g, b):
    mu = x.mean(-1, keepdims=True)
    var = ((x - mu) ** 2).mean(-1, keepdims=True)
    return (x - mu) / jnp.sqrt(var + 1e-5) * g + b


def _silu(x):
    return x * jax.nn.sigmoid(x)


def _layer_body(x_ref, *refs, L, D, r, Cn, Q, final_ln):
    N = N_STATE
    Lp = Cn * Q
    (wixT1, wizT1, cw01, cw11, cb1, wxdT1, wxbT1, wxcT1, wdtT1, bdt1, anT1, dv1, woT1,
     wixT2, wizT2, cw02, cw12, cb2, wxdT2, wxbT2, wxcT2, wdtT2, bdt2, anT2, dv2, woT2,
     ln1g, ln1b, ffw1T, ffb1, ffw2T, ffb2, ln2g, ln2b, fg, fb,
     o_ref, dt_s, dtxc_s, bm_s, cm_s) = refs

    LB = Lp * B
    pads = [divmod(t, Q) for t in range(L, Lp)]
    x = x_ref[:]
    x2 = x.reshape(LB, D)

    def run_dir(wixT, wizT, cw0, cw1, cb, wxdT, wxbT, wxcT, wdtT, bdt, anT, dv,
                woT, reverse):
        xp = jnp.dot(x2, wixT[:], preferred_element_type=jnp.float32)
        z = jnp.dot(x2, wizT[:], preferred_element_type=jnp.float32)
        xp3 = xp.reshape(Lp, B, D)
        zero = jnp.zeros((1, B, D), jnp.float32)
        if Cn == 1:
            if not reverse:
                xsh = jnp.concatenate([zero, xp3[:-1]], axis=0)
            else:
                xsh = jnp.concatenate([xp3[1:], zero], axis=0)
        elif not reverse:
            first = jnp.concatenate(
                [zero, xp3[(Q - 1) * Cn:(Q - 1) * Cn + Cn - 1]], axis=0)
            xsh = jnp.concatenate([first, xp3[:(Q - 1) * Cn]], axis=0)
        else:
            last = jnp.concatenate([xp3[1:Cn], zero], axis=0)
            xsh = jnp.concatenate([xp3[Cn:], last], axis=0)
        xc = xsh * cw0[:] + xp3 * cw1[:] + cb[:]
        xc = _silu(xc)
        xc2 = xc.reshape(LB, D)
        dtl = jnp.dot(xc2, wxdT[:], preferred_element_type=jnp.float32)
        bm = jnp.dot(xc2, wxbT[:], preferred_element_type=jnp.float32)
        cm = jnp.dot(xc2, wxcT[:], preferred_element_type=jnp.float32)
        dtf = jax.nn.softplus(
            jnp.dot(dtl, wdtT[:], preferred_element_type=jnp.float32) + bdt[:])
        dt_s[:] = dtf.reshape(Q, Cn, B, D)
        dtxc_s[:] = (dtf * xc2).reshape(Q, Cn, B, D)
        bm_s[:] = bm.reshape(Q, Cn, B, N).astype(jnp.bfloat16)
        cm_s[:] = cm.reshape(Q, Cn, B, N).astype(jnp.bfloat16)
        for (k, j) in pads:
            dt_s[j, k] = jnp.zeros((B, D), jnp.float32)
            dtxc_s[j, k] = jnp.zeros((B, D), jnp.float32)
        anT_v = anT[:]

        def dA_of(dtq):
            return jnp.exp(dtq[:, :, None, :] * anT_v[None, None, :, :])

        def dBx_of(dtxcq, bq):
            return dtxcq[:, :, None, :] * bq[:, :, :, None]

        def jj_of(i):
            return Q - 1 - i if reverse else i

        if Cn == 1:
            U = 4 if Q % 4 == 0 else (2 if Q % 2 == 0 else 1)

            def substep(jj, h):
                dtq = dt_s[jj]
                h = (dA_of(dtq) * h
                     + dBx_of(dtxc_s[jj], bm_s[jj].astype(jnp.float32)))
                dt_s[jj] = jnp.sum(
                    h * cm_s[jj].astype(jnp.float32)[:, :, :, None], axis=2)
                return h

            def step(i, h):
                base = i * U
                for u in range(U):
                    jj = jj_of(base + u)
                    h = substep(jj, h)
                return h
            lax.fori_loop(0, Q // U, step,
                          jnp.zeros((Cn, B, N, D), jnp.float32))
        else:
            def step_a(i, carry):
                h, sdt = carry
                jj = jj_of(i)
                dtq = dt_s[jj]
                h = dA_of(dtq) * h + dBx_of(dtxc_s[jj], bm_s[jj])
                return h, sdt + dtq
            S, sdt = lax.fori_loop(
                0, Q, step_a,
                (jnp.zeros((Cn, B, N, D), jnp.float32),
                 jnp.zeros((Cn, B, D), jnp.float32)))
            P = jnp.exp(sdt[:, :, None, :] * anT_v[None, None, :, :])
            g = jnp.zeros((B, N, D), jnp.float32)
            gs = [None] * Cn
            order = range(Cn - 1, -1, -1) if reverse else range(Cn)
            for k in order:
                gs[k] = g
                g = P[k] * g + S[k]
            G = jnp.stack(gs)

            def step_c(i, h):
                jj = jj_of(i)
                dtq = dt_s[jj]
                h = dA_of(dtq) * h + dBx_of(dtxc_s[jj], bm_s[jj])
                dt_s[jj] = jnp.sum(h * cm_s[jj][:, :, :, None], axis=2)
                return h
            lax.fori_loop(0, Q, step_c, G)

        y = dt_s[:].reshape(Lp, B, D) + xc * dv[:]
        y = y * _silu(z.reshape(Lp, B, D))
        return jnp.dot(y.reshape(LB, D), woT[:], preferred_element_type=jnp.float32)

    o1 = run_dir(wixT1, wizT1, cw01, cw11, cb1, wxdT1, wxbT1, wxcT1, wdtT1,
                 bdt1, anT1, dv1, woT1, reverse=False)
    o2 = run_dir(wixT2, wizT2, cw02, cw12, cb2, wxdT2, wxbT2, wxcT2, wdtT2,
                 bdt2, anT2, dv2, woT2, reverse=True)

    xr = x + (o1 + o2).reshape(Lp, B, D)
    x1 = _ln(xr, ln1g[:], ln1b[:])
    x1f = x1.reshape(LB, D)
    halves = []
    for ci in range(2):
        seg = x1f[ci * (LB // 2):(ci + 1) * (LB // 2)]
        fh = jax.nn.gelu(
            jnp.dot(seg, ffw1T[:], preferred_element_type=jnp.float32)
            + ffb1[:])
        halves.append(jnp.dot(fh, ffw2T[:], preferred_element_type=jnp.float32))
    ffo = jnp.concatenate(halves, axis=0) + ffb2[:]
    out = _ln(x1 + ffo.reshape(Lp, B, D), ln2g[:], ln2b[:])
    if final_ln:
        out = _ln(out, fg[:], fb[:])
    o_ref[:] = out
    for (k, j) in pads:
        o_ref[j * Cn + k] = jnp.zeros((B, D), jnp.float32)


def _prep_mamba(p):
    r = p['W_dt'].shape[1]
    d = p['W_in'].shape[1]
    wiT = p['W_in'].T
    wxT = p['W_x'].T
    return (
        wiT[:, :d], wiT[:, d:],
        p['conv_w'][:, 0], p['conv_w'][:, 1], p['conv_b'],
        wxT[:, :r], wxT[:, r:r + N_STATE], wxT[:, r + N_STATE:],
        p['W_dt'].T, p['b_dt'],
        (-jnp.exp(p['A_log'])).T,
        p['D'], p['W_out'].T,
    )


def _layer_call(x_lbd, lp, fin_g, fin_b, L, D, r, Cn, Q, final_ln,
                interpret=False):
    N = N_STATE
    Lp = Cn * Q
    args = (x_lbd, *_prep_mamba(lp['m1']), *_prep_mamba(lp['m2']),
            lp['ln1_g'], lp['ln1_b'], lp['ff_w1'].T, lp['ff_b1'],
            lp['ff_w2'].T, lp['ff_b2'], lp['ln2_g'], lp['ln2_b'], fin_g, fin_b)
    return pl.pallas_call(
        functools.partial(_layer_body, L=L, D=D, r=r, Cn=Cn, Q=Q,
                          final_ln=final_ln),
        out_shape=jax.ShapeDtypeStruct((Lp, B, D), jnp.float32),
        scratch_shapes=[
            pltpu.VMEM((Q, Cn, B, D), jnp.float32),
            pltpu.VMEM((Q, Cn, B, D), jnp.float32),
            pltpu.VMEM((Q, Cn, B, N), jnp.bfloat16),
            pltpu.VMEM((Q, Cn, B, N), jnp.bfloat16),
        ],
        interpret=interpret,
    )(*args)


def _interleave(x_nat, Cn, Q):
    L = x_nat.shape[0]
    Lp = Cn * Q
    if Lp > L:
        x_nat = jnp.concatenate(
            [x_nat, jnp.zeros((Lp - L,) + x_nat.shape[1:], x_nat.dtype)], axis=0)
    return x_nat.reshape(Cn, Q, *x_nat.shape[1:]).swapaxes(0, 1).reshape(
        Lp, *x_nat.shape[1:])


def _deinterleave(y, Cn, Q, L):
    y = y.reshape(Q, Cn, *y.shape[1:]).swapaxes(0, 1).reshape(
        Cn * Q, *y.shape[1:])
    return y[:L]


def _encoder(x_lbd, ep, L, D, r, Cn, Q, interpret=False):
    h = _interleave(x_lbd, Cn, Q)
    for i, lp in enumerate(ep['layers']):
        h = _layer_call(h, lp, ep['ln_g'], ep['ln_b'], L, D, r, Cn, Q,
                        final_ln=(i == DEPTH - 1), interpret=interpret)
    return _deinterleave(h, Cn, Q, L)


def _gate_body(x_ref, gwT_ref, gb_ref, i1_ref, i2_ref):
    xv = x_ref[:]
    pooled = jnp.mean(xv, axis=1)
    logits = jnp.dot(pooled, gwT_ref[:], preferred_element_type=jnp.float32)
    logits = logits + gb_ref[:]
    mx = jnp.max(logits, axis=-1, keepdims=True)
    i4 = lax.broadcasted_iota(jnp.int32, logits.shape, 1)
    e = jnp.min(jnp.where(logits >= mx, i4, 4), axis=-1, keepdims=True)
    first_T = ((e == 0) | (e == 3)).astype(jnp.int32)
    second_S = ((e == 0) | (e == 2)).astype(jnp.int32)
    biota = lax.broadcasted_iota(jnp.int32, (B, 1), 0)
    i1_ref[:] = biota + B * (1 - first_T)
    i2_ref[:] = biota + B * (1 - second_S)


def _gate(x, gw, gb, interpret=False):
    i1, i2 = pl.pallas_call(
        _gate_body,
        out_shape=(jax.ShapeDtypeStruct((B, 1), jnp.int32),
                   jax.ShapeDtypeStruct((B, 1), jnp.int32)),
        interpret=interpret,
    )(x, gw.T, gb)
    return i1.reshape(B), i2.reshape(B)


_B_PER_W = 8
_N_SEL_W = B // _B_PER_W


def _make_select():
    mesh = plsc.VectorSubcoreMesh(core_axis_name="c", subcore_axis_name="s")

    @functools.partial(
        pl.kernel, mesh=mesh,
        out_type=jax.ShapeDtypeStruct((B, ROW), jnp.float32),
        scratch_types=[
            pltpu.VMEM((_B_PER_W,), jnp.int32),
            pltpu.VMEM((_B_PER_W, ROW), jnp.float32),
            pltpu.SemaphoreType.DMA,
        ],
    )
    def sel(table_hbm, idx_hbm, out_hbm, idx_v, rows_v, sem):
        wid = lax.axis_index("s") * 2 + lax.axis_index("c")

        @pl.when(wid < _N_SEL_W)
        def _():
            base = wid * _B_PER_W
            pltpu.sync_copy(idx_hbm.at[pl.ds(base, _B_PER_W)], idx_v)
            pltpu.async_copy(table_hbm.at[idx_v], rows_v, sem).wait()
            pltpu.sync_copy(rows_v, out_hbm.at[pl.ds(base, _B_PER_W)])

    return sel


_SEL_CACHE = []


def _select(table, idx):
    if not _SEL_CACHE:
        _SEL_CACHE.append(_make_select())
    return _SEL_CACHE[0](table, idx)


def kernel(x, params):
    pT = params['enc_T']
    pS = params['enc_S']
    rT = pT['layers'][0]['m1']['W_dt'].shape[1]
    rS = pS['layers'][0]['m1']['W_dt'].shape[1]

    encT = functools.partial(_encoder, ep=pT, L=T, D=J3, r=rT, Cn=1, Q=T)
    encS = functools.partial(_encoder, ep=pS, L=J3, D=T, r=rS, Cn=1, Q=J3)

    u = encT(x.transpose(2, 0, 1))
    v = encS(x.transpose(1, 0, 2))

    i1, i2 = _gate(x, params['gate_w'], params['gate_b'])

    u_rows = u.transpose(1, 2, 0).reshape(B, ROW)
    v_rows = v.transpose(1, 0, 2).reshape(B, ROW)
    y1 = _select(jnp.concatenate([u_rows, v_rows], axis=0), i1)
    y1 = y1.reshape(B, J3, T)

    a = encS(y1.transpose(1, 0, 2))
    c = encT(y1.transpose(2, 0, 1))
    a_rows = a.transpose(1, 0, 2).reshape(B, ROW)
    c_rows = c.transpose(1, 2, 0).reshape(B, ROW)
    out = _select(jnp.concatenate([a_rows, c_rows], axis=0), i2)
    return out.reshape(B, J3, T)


# speedup vs baseline: 1.7654x; 1.0328x over previous
import functools
import math

import jax
import jax.numpy as jnp
from jax import lax
from jax.experimental import pallas as pl
from jax.experimental.pallas import tpu as pltpu
from jax.experimental.pallas import tpu_sc as plsc

B = 32
J3 = 34
T = 256
N_STATE = 32
DEPTH = 3
ROW = J3 * T


def _ln(x, g, b):
    mu = x.mean(-1, keepdims=True)
    var = ((x - mu) ** 2).mean(-1, keepdims=True)
    return (x - mu) / jnp.sqrt(var + 1e-5) * g + b


def _silu(x):
    return x * jax.nn.sigmoid(x)


def _layer_body(x_ref, *refs, L, D, r, Cn, Q, final_ln):
    N = N_STATE
    Lp = Cn * Q
    (wixT1, wizT1, cw01, cw11, cb1, wxdT1, wxbT1, wxcT1, wdtT1, bdt1, anT1, dv1, woT1,
     wixT2, wizT2, cw02, cw12, cb2, wxdT2, wxbT2, wxcT2, wdtT2, bdt2, anT2, dv2, woT2,
     ln1g, ln1b, ffw1T, ffb1, ffw2T, ffb2, ln2g, ln2b, fg, fb,
     o_ref, dt_s, dtxc_s, bm_s, cm_s) = refs

    LB = Lp * B
    pads = [divmod(t, Q) for t in range(L, Lp)]
    x = x_ref[:]
    x2 = x.reshape(LB, D)

    def run_dir(wixT, wizT, cw0, cw1, cb, wxdT, wxbT, wxcT, wdtT, bdt, anT, dv,
                woT, reverse):
        xp = jnp.dot(x2, wixT[:], preferred_element_type=jnp.float32)
        z = jnp.dot(x2, wizT[:], preferred_element_type=jnp.float32)
        xp3 = xp.reshape(Lp, B, D)
        zero = jnp.zeros((1, B, D), jnp.float32)
        if Cn == 1:
            if not reverse:
                xsh = jnp.concatenate([zero, xp3[:-1]], axis=0)
            else:
                xsh = jnp.concatenate([xp3[1:], zero], axis=0)
        elif not reverse:
            first = jnp.concatenate(
                [zero, xp3[(Q - 1) * Cn:(Q - 1) * Cn + Cn - 1]], axis=0)
            xsh = jnp.concatenate([first, xp3[:(Q - 1) * Cn]], axis=0)
        else:
            last = jnp.concatenate([xp3[1:Cn], zero], axis=0)
            xsh = jnp.concatenate([xp3[Cn:], last], axis=0)
        xc = xsh * cw0[:] + xp3 * cw1[:] + cb[:]
        xc = _silu(xc)
        xc2 = xc.reshape(LB, D)
        dtl = jnp.dot(xc2, wxdT[:], preferred_element_type=jnp.float32)
        bm = jnp.dot(xc2, wxbT[:], preferred_element_type=jnp.float32)
        cm = jnp.dot(xc2, wxcT[:], preferred_element_type=jnp.float32)
        dtf = jax.nn.softplus(
            jnp.dot(dtl, wdtT[:], preferred_element_type=jnp.float32) + bdt[:])
        dt_s[:] = dtf.reshape(Q, Cn, B, D)
        dtxc_s[:] = (dtf * xc2).reshape(Q, Cn, B, D)
        bm_s[:] = bm.reshape(Q, Cn, B, N).astype(jnp.bfloat16)
        cm_s[:] = cm.reshape(Q, Cn, B, N).astype(jnp.bfloat16)
        for (k, j) in pads:
            dt_s[j, k] = jnp.zeros((B, D), jnp.float32)
            dtxc_s[j, k] = jnp.zeros((B, D), jnp.float32)
        anT_v = anT[:]

        def dA_of(dtq):
            return jnp.exp(dtq[:, :, None, :] * anT_v[None, None, :, :])

        def dBx_of(dtxcq, bq):
            return dtxcq[:, :, None, :] * bq[:, :, :, None]

        def jj_of(i):
            return Q - 1 - i if reverse else i

        if Cn == 1:
            U = 8 if Q % 8 == 0 else (2 if Q % 2 == 0 else 1)

            def substep(jj, h):
                dtq = dt_s[jj]
                h = (dA_of(dtq) * h
                     + dBx_of(dtxc_s[jj], bm_s[jj].astype(jnp.float32)))
                dt_s[jj] = jnp.sum(
                    h * cm_s[jj].astype(jnp.float32)[:, :, :, None], axis=2)
                return h

            def step(i, h):
                base = i * U
                for u in range(U):
                    jj = jj_of(base + u)
                    h = substep(jj, h)
                return h
            lax.fori_loop(0, Q // U, step,
                          jnp.zeros((Cn, B, N, D), jnp.float32))
        else:
            def step_a(i, carry):
                h, sdt = carry
                jj = jj_of(i)
                dtq = dt_s[jj]
                h = dA_of(dtq) * h + dBx_of(dtxc_s[jj], bm_s[jj])
                return h, sdt + dtq
            S, sdt = lax.fori_loop(
                0, Q, step_a,
                (jnp.zeros((Cn, B, N, D), jnp.float32),
                 jnp.zeros((Cn, B, D), jnp.float32)))
            P = jnp.exp(sdt[:, :, None, :] * anT_v[None, None, :, :])
            g = jnp.zeros((B, N, D), jnp.float32)
            gs = [None] * Cn
            order = range(Cn - 1, -1, -1) if reverse else range(Cn)
            for k in order:
                gs[k] = g
                g = P[k] * g + S[k]
            G = jnp.stack(gs)

            def step_c(i, h):
                jj = jj_of(i)
                dtq = dt_s[jj]
                h = dA_of(dtq) * h + dBx_of(dtxc_s[jj], bm_s[jj])
                dt_s[jj] = jnp.sum(h * cm_s[jj][:, :, :, None], axis=2)
                return h
            lax.fori_loop(0, Q, step_c, G)

        y = dt_s[:].reshape(Lp, B, D) + xc * dv[:]
        y = y * _silu(z.reshape(Lp, B, D))
        return jnp.dot(y.reshape(LB, D), woT[:], preferred_element_type=jnp.float32)

    o1 = run_dir(wixT1, wizT1, cw01, cw11, cb1, wxdT1, wxbT1, wxcT1, wdtT1,
                 bdt1, anT1, dv1, woT1, reverse=False)
    o2 = run_dir(wixT2, wizT2, cw02, cw12, cb2, wxdT2, wxbT2, wxcT2, wdtT2,
                 bdt2, anT2, dv2, woT2, reverse=True)

    xr = x + (o1 + o2).reshape(Lp, B, D)
    x1 = _ln(xr, ln1g[:], ln1b[:])
    x1f = x1.reshape(LB, D)
    halves = []
    for ci in range(2):
        seg = x1f[ci * (LB // 2):(ci + 1) * (LB // 2)]
        fh = jax.nn.gelu(
            jnp.dot(seg, ffw1T[:], preferred_element_type=jnp.float32)
            + ffb1[:])
        halves.append(jnp.dot(fh, ffw2T[:], preferred_element_type=jnp.float32))
    ffo = jnp.concatenate(halves, axis=0) + ffb2[:]
    out = _ln(x1 + ffo.reshape(Lp, B, D), ln2g[:], ln2b[:])
    if final_ln:
        out = _ln(out, fg[:], fb[:])
    o_ref[:] = out
    for (k, j) in pads:
        o_ref[j * Cn + k] = jnp.zeros((B, D), jnp.float32)


def _prep_mamba(p):
    r = p['W_dt'].shape[1]
    d = p['W_in'].shape[1]
    wiT = p['W_in'].T
    wxT = p['W_x'].T
    return (
        wiT[:, :d], wiT[:, d:],
        p['conv_w'][:, 0], p['conv_w'][:, 1], p['conv_b'],
        wxT[:, :r], wxT[:, r:r + N_STATE], wxT[:, r + N_STATE:],
        p['W_dt'].T, p['b_dt'],
        (-jnp.exp(p['A_log'])).T,
        p['D'], p['W_out'].T,
    )


def _layer_call(x_lbd, lp, fin_g, fin_b, L, D, r, Cn, Q, final_ln,
                interpret=False):
    N = N_STATE
    Lp = Cn * Q
    args = (x_lbd, *_prep_mamba(lp['m1']), *_prep_mamba(lp['m2']),
            lp['ln1_g'], lp['ln1_b'], lp['ff_w1'].T, lp['ff_b1'],
            lp['ff_w2'].T, lp['ff_b2'], lp['ln2_g'], lp['ln2_b'], fin_g, fin_b)
    return pl.pallas_call(
        functools.partial(_layer_body, L=L, D=D, r=r, Cn=Cn, Q=Q,
                          final_ln=final_ln),
        out_shape=jax.ShapeDtypeStruct((Lp, B, D), jnp.float32),
        scratch_shapes=[
            pltpu.VMEM((Q, Cn, B, D), jnp.float32),
            pltpu.VMEM((Q, Cn, B, D), jnp.float32),
            pltpu.VMEM((Q, Cn, B, N), jnp.bfloat16),
            pltpu.VMEM((Q, Cn, B, N), jnp.bfloat16),
        ],
        interpret=interpret,
    )(*args)


def _interleave(x_nat, Cn, Q):
    L = x_nat.shape[0]
    Lp = Cn * Q
    if Lp > L:
        x_nat = jnp.concatenate(
            [x_nat, jnp.zeros((Lp - L,) + x_nat.shape[1:], x_nat.dtype)], axis=0)
    return x_nat.reshape(Cn, Q, *x_nat.shape[1:]).swapaxes(0, 1).reshape(
        Lp, *x_nat.shape[1:])


def _deinterleave(y, Cn, Q, L):
    y = y.reshape(Q, Cn, *y.shape[1:]).swapaxes(0, 1).reshape(
        Cn * Q, *y.shape[1:])
    return y[:L]


def _encoder(x_lbd, ep, L, D, r, Cn, Q, interpret=False):
    h = _interleave(x_lbd, Cn, Q)
    for i, lp in enumerate(ep['layers']):
        h = _layer_call(h, lp, ep['ln_g'], ep['ln_b'], L, D, r, Cn, Q,
                        final_ln=(i == DEPTH - 1), interpret=interpret)
    return _deinterleave(h, Cn, Q, L)


def _gate_body(x_ref, gwT_ref, gb_ref, i1_ref, i2_ref):
    xv = x_ref[:]
    pooled = jnp.mean(xv, axis=1)
    logits = jnp.dot(pooled, gwT_ref[:], preferred_element_type=jnp.float32)
    logits = logits + gb_ref[:]
    mx = jnp.max(logits, axis=-1, keepdims=True)
    i4 = lax.broadcasted_iota(jnp.int32, logits.shape, 1)
    e = jnp.min(jnp.where(logits >= mx, i4, 4), axis=-1, keepdims=True)
    first_T = ((e == 0) | (e == 3)).astype(jnp.int32)
    second_S = ((e == 0) | (e == 2)).astype(jnp.int32)
    biota = lax.broadcasted_iota(jnp.int32, (B, 1), 0)
    i1_ref[:] = biota + B * (1 - first_T)
    i2_ref[:] = biota + B * (1 - second_S)


def _gate(x, gw, gb, interpret=False):
    i1, i2 = pl.pallas_call(
        _gate_body,
        out_shape=(jax.ShapeDtypeStruct((B, 1), jnp.int32),
                   jax.ShapeDtypeStruct((B, 1), jnp.int32)),
        interpret=interpret,
    )(x, gw.T, gb)
    return i1.reshape(B), i2.reshape(B)


_B_PER_W = 8
_N_SEL_W = B // _B_PER_W


def _make_select():
    mesh = plsc.VectorSubcoreMesh(core_axis_name="c", subcore_axis_name="s")

    @functools.partial(
        pl.kernel, mesh=mesh,
        out_type=jax.ShapeDtypeStruct((B, ROW), jnp.float32),
        scratch_types=[
            pltpu.VMEM((_B_PER_W,), jnp.int32),
            pltpu.VMEM((_B_PER_W, ROW), jnp.float32),
            pltpu.SemaphoreType.DMA,
        ],
    )
    def sel(table_hbm, idx_hbm, out_hbm, idx_v, rows_v, sem):
        wid = lax.axis_index("s") * 2 + lax.axis_index("c")

        @pl.when(wid < _N_SEL_W)
        def _():
            base = wid * _B_PER_W
            pltpu.sync_copy(idx_hbm.at[pl.ds(base, _B_PER_W)], idx_v)
            pltpu.async_copy(table_hbm.at[idx_v], rows_v, sem).wait()
            pltpu.sync_copy(rows_v, out_hbm.at[pl.ds(base, _B_PER_W)])

    return sel


_SEL_CACHE = []


def _select(table, idx):
    if not _SEL_CACHE:
        _SEL_CACHE.append(_make_select())
    return _SEL_CACHE[0](table, idx)


def kernel(x, params):
    pT = params['enc_T']
    pS = params['enc_S']
    rT = pT['layers'][0]['m1']['W_dt'].shape[1]
    rS = pS['layers'][0]['m1']['W_dt'].shape[1]

    encT = functools.partial(_encoder, ep=pT, L=T, D=J3, r=rT, Cn=1, Q=T)
    encS = functools.partial(_encoder, ep=pS, L=J3, D=T, r=rS, Cn=1, Q=J3)

    u = encT(x.transpose(2, 0, 1))
    v = encS(x.transpose(1, 0, 2))

    i1, i2 = _gate(x, params['gate_w'], params['gate_b'])

    u_rows = u.transpose(1, 2, 0).reshape(B, ROW)
    v_rows = v.transpose(1, 0, 2).reshape(B, ROW)
    y1 = _select(jnp.concatenate([u_rows, v_rows], axis=0), i1)
    y1 = y1.reshape(B, J3, T)

    a = encS(y1.transpose(1, 0, 2))
    c = encT(y1.transpose(2, 0, 1))
    a_rows = a.transpose(1, 0, 2).reshape(B, ROW)
    c_rows = c.transpose(1, 2, 0).reshape(B, ROW)
    out = _select(jnp.concatenate([a_rows, c_rows], axis=0), i2)
    return out.reshape(B, J3, T)


# speedup vs baseline: 1.7968x; 1.0178x over previous
import functools
import math

import jax
import jax.numpy as jnp
from jax import lax
from jax.experimental import pallas as pl
from jax.experimental.pallas import tpu as pltpu
from jax.experimental.pallas import tpu_sc as plsc

B = 32
J3 = 34
T = 256
N_STATE = 32
DEPTH = 3
ROW = J3 * T


def _ln(x, g, b):
    mu = x.mean(-1, keepdims=True)
    var = ((x - mu) ** 2).mean(-1, keepdims=True)
    return (x - mu) / jnp.sqrt(var + 1e-5) * g + b


def _silu(x):
    return x * jax.nn.sigmoid(x)


def _layer_body(x_ref, *refs, L, D, r, Cn, Q, final_ln):
    N = N_STATE
    Lp = Cn * Q
    (wixT1, wizT1, cw01, cw11, cb1, wxdT1, wxbT1, wxcT1, wdtT1, bdt1, anT1, dv1, woT1,
     wixT2, wizT2, cw02, cw12, cb2, wxdT2, wxbT2, wxcT2, wdtT2, bdt2, anT2, dv2, woT2,
     ln1g, ln1b, ffw1T, ffb1, ffw2T, ffb2, ln2g, ln2b, fg, fb,
     o_ref, dt_s, dtxc_s, bm_s, cm_s) = refs

    LB = Lp * B
    pads = [divmod(t, Q) for t in range(L, Lp)]
    x = x_ref[:]
    x2 = x.reshape(LB, D)

    def run_dir(wixT, wizT, cw0, cw1, cb, wxdT, wxbT, wxcT, wdtT, bdt, anT, dv,
                woT, reverse):
        xp = jnp.dot(x2, wixT[:], preferred_element_type=jnp.float32)
        z = jnp.dot(x2, wizT[:], preferred_element_type=jnp.float32)
        xp3 = xp.reshape(Lp, B, D)
        zero = jnp.zeros((1, B, D), jnp.float32)
        if Cn == 1:
            if not reverse:
                xsh = jnp.concatenate([zero, xp3[:-1]], axis=0)
            else:
                xsh = jnp.concatenate([xp3[1:], zero], axis=0)
        elif not reverse:
            first = jnp.concatenate(
                [zero, xp3[(Q - 1) * Cn:(Q - 1) * Cn + Cn - 1]], axis=0)
            xsh = jnp.concatenate([first, xp3[:(Q - 1) * Cn]], axis=0)
        else:
            last = jnp.concatenate([xp3[1:Cn], zero], axis=0)
            xsh = jnp.concatenate([xp3[Cn:], last], axis=0)
        xc = xsh * cw0[:] + xp3 * cw1[:] + cb[:]
        xc = _silu(xc)
        xc2 = xc.reshape(LB, D)
        dtl = jnp.dot(xc2, wxdT[:], preferred_element_type=jnp.float32)
        bm = jnp.dot(xc2, wxbT[:], preferred_element_type=jnp.float32)
        cm = jnp.dot(xc2, wxcT[:], preferred_element_type=jnp.float32)
        dtf = jax.nn.softplus(
            jnp.dot(dtl, wdtT[:], preferred_element_type=jnp.float32) + bdt[:])
        dt_s[:] = dtf.reshape(Q, Cn, B, D)
        dtxc_s[:] = (dtf * xc2).reshape(Q, Cn, B, D)
        bm_s[:] = bm.reshape(Q, Cn, B, N).astype(jnp.bfloat16)
        cm_s[:] = cm.reshape(Q, Cn, B, N).astype(jnp.bfloat16)
        for (k, j) in pads:
            dt_s[j, k] = jnp.zeros((B, D), jnp.float32)
            dtxc_s[j, k] = jnp.zeros((B, D), jnp.float32)
        anT_v = anT[:]

        def dA_of(dtq):
            return jnp.exp(dtq[:, :, None, :] * anT_v[None, None, :, :])

        def dBx_of(dtxcq, bq):
            return dtxcq[:, :, None, :] * bq[:, :, :, None]

        def jj_of(i):
            return Q - 1 - i if reverse else i

        if Cn == 1:
            U = 16 if Q % 16 == 0 else (2 if Q % 2 == 0 else 1)

            def substep(jj, h):
                dtq = dt_s[jj]
                h = (dA_of(dtq) * h
                     + dBx_of(dtxc_s[jj], bm_s[jj].astype(jnp.float32)))
                dt_s[jj] = jnp.sum(
                    h * cm_s[jj].astype(jnp.float32)[:, :, :, None], axis=2)
                return h

            def step(i, h):
                base = i * U
                for u in range(U):
                    jj = jj_of(base + u)
                    h = substep(jj, h)
                return h
            lax.fori_loop(0, Q // U, step,
                          jnp.zeros((Cn, B, N, D), jnp.float32))
        else:
            def step_a(i, carry):
                h, sdt = carry
                jj = jj_of(i)
                dtq = dt_s[jj]
                h = dA_of(dtq) * h + dBx_of(dtxc_s[jj], bm_s[jj])
                return h, sdt + dtq
            S, sdt = lax.fori_loop(
                0, Q, step_a,
                (jnp.zeros((Cn, B, N, D), jnp.float32),
                 jnp.zeros((Cn, B, D), jnp.float32)))
            P = jnp.exp(sdt[:, :, None, :] * anT_v[None, None, :, :])
            g = jnp.zeros((B, N, D), jnp.float32)
            gs = [None] * Cn
            order = range(Cn - 1, -1, -1) if reverse else range(Cn)
            for k in order:
                gs[k] = g
                g = P[k] * g + S[k]
            G = jnp.stack(gs)

            def step_c(i, h):
                jj = jj_of(i)
                dtq = dt_s[jj]
                h = dA_of(dtq) * h + dBx_of(dtxc_s[jj], bm_s[jj])
                dt_s[jj] = jnp.sum(h * cm_s[jj][:, :, :, None], axis=2)
                return h
            lax.fori_loop(0, Q, step_c, G)

        y = dt_s[:].reshape(Lp, B, D) + xc * dv[:]
        y = y * _silu(z.reshape(Lp, B, D))
        return jnp.dot(y.reshape(LB, D), woT[:], preferred_element_type=jnp.float32)

    o1 = run_dir(wixT1, wizT1, cw01, cw11, cb1, wxdT1, wxbT1, wxcT1, wdtT1,
                 bdt1, anT1, dv1, woT1, reverse=False)
    o2 = run_dir(wixT2, wizT2, cw02, cw12, cb2, wxdT2, wxbT2, wxcT2, wdtT2,
                 bdt2, anT2, dv2, woT2, reverse=True)

    xr = x + (o1 + o2).reshape(Lp, B, D)
    x1 = _ln(xr, ln1g[:], ln1b[:])
    x1f = x1.reshape(LB, D)
    halves = []
    for ci in range(2):
        seg = x1f[ci * (LB // 2):(ci + 1) * (LB // 2)]
        fh = jax.nn.gelu(
            jnp.dot(seg, ffw1T[:], preferred_element_type=jnp.float32)
            + ffb1[:])
        halves.append(jnp.dot(fh, ffw2T[:], preferred_element_type=jnp.float32))
    ffo = jnp.concatenate(halves, axis=0) + ffb2[:]
    out = _ln(x1 + ffo.reshape(Lp, B, D), ln2g[:], ln2b[:])
    if final_ln:
        out = _ln(out, fg[:], fb[:])
    o_ref[:] = out
    for (k, j) in pads:
        o_ref[j * Cn + k] = jnp.zeros((B, D), jnp.float32)


def _prep_mamba(p):
    r = p['W_dt'].shape[1]
    d = p['W_in'].shape[1]
    wiT = p['W_in'].T
    wxT = p['W_x'].T
    return (
        wiT[:, :d], wiT[:, d:],
        p['conv_w'][:, 0], p['conv_w'][:, 1], p['conv_b'],
        wxT[:, :r], wxT[:, r:r + N_STATE], wxT[:, r + N_STATE:],
        p['W_dt'].T, p['b_dt'],
        (-jnp.exp(p['A_log'])).T,
        p['D'], p['W_out'].T,
    )


def _layer_call(x_lbd, lp, fin_g, fin_b, L, D, r, Cn, Q, final_ln,
                interpret=False):
    N = N_STATE
    Lp = Cn * Q
    args = (x_lbd, *_prep_mamba(lp['m1']), *_prep_mamba(lp['m2']),
            lp['ln1_g'], lp['ln1_b'], lp['ff_w1'].T, lp['ff_b1'],
            lp['ff_w2'].T, lp['ff_b2'], lp['ln2_g'], lp['ln2_b'], fin_g, fin_b)
    return pl.pallas_call(
        functools.partial(_layer_body, L=L, D=D, r=r, Cn=Cn, Q=Q,
                          final_ln=final_ln),
        out_shape=jax.ShapeDtypeStruct((Lp, B, D), jnp.float32),
        scratch_shapes=[
            pltpu.VMEM((Q, Cn, B, D), jnp.float32),
            pltpu.VMEM((Q, Cn, B, D), jnp.float32),
            pltpu.VMEM((Q, Cn, B, N), jnp.bfloat16),
            pltpu.VMEM((Q, Cn, B, N), jnp.bfloat16),
        ],
        interpret=interpret,
    )(*args)


def _interleave(x_nat, Cn, Q):
    L = x_nat.shape[0]
    Lp = Cn * Q
    if Lp > L:
        x_nat = jnp.concatenate(
            [x_nat, jnp.zeros((Lp - L,) + x_nat.shape[1:], x_nat.dtype)], axis=0)
    return x_nat.reshape(Cn, Q, *x_nat.shape[1:]).swapaxes(0, 1).reshape(
        Lp, *x_nat.shape[1:])


def _deinterleave(y, Cn, Q, L):
    y = y.reshape(Q, Cn, *y.shape[1:]).swapaxes(0, 1).reshape(
        Cn * Q, *y.shape[1:])
    return y[:L]


def _encoder(x_lbd, ep, L, D, r, Cn, Q, interpret=False):
    h = _interleave(x_lbd, Cn, Q)
    for i, lp in enumerate(ep['layers']):
        h = _layer_call(h, lp, ep['ln_g'], ep['ln_b'], L, D, r, Cn, Q,
                        final_ln=(i == DEPTH - 1), interpret=interpret)
    return _deinterleave(h, Cn, Q, L)


def _gate_body(x_ref, gwT_ref, gb_ref, i1_ref, i2_ref):
    xv = x_ref[:]
    pooled = jnp.mean(xv, axis=1)
    logits = jnp.dot(pooled, gwT_ref[:], preferred_element_type=jnp.float32)
    logits = logits + gb_ref[:]
    mx = jnp.max(logits, axis=-1, keepdims=True)
    i4 = lax.broadcasted_iota(jnp.int32, logits.shape, 1)
    e = jnp.min(jnp.where(logits >= mx, i4, 4), axis=-1, keepdims=True)
    first_T = ((e == 0) | (e == 3)).astype(jnp.int32)
    second_S = ((e == 0) | (e == 2)).astype(jnp.int32)
    biota = lax.broadcasted_iota(jnp.int32, (B, 1), 0)
    i1_ref[:] = biota + B * (1 - first_T)
    i2_ref[:] = biota + B * (1 - second_S)


def _gate(x, gw, gb, interpret=False):
    i1, i2 = pl.pallas_call(
        _gate_body,
        out_shape=(jax.ShapeDtypeStruct((B, 1), jnp.int32),
                   jax.ShapeDtypeStruct((B, 1), jnp.int32)),
        interpret=interpret,
    )(x, gw.T, gb)
    return i1.reshape(B), i2.reshape(B)


_B_PER_W = 8
_N_SEL_W = B // _B_PER_W


def _make_select():
    mesh = plsc.VectorSubcoreMesh(core_axis_name="c", subcore_axis_name="s")

    @functools.partial(
        pl.kernel, mesh=mesh,
        out_type=jax.ShapeDtypeStruct((B, ROW), jnp.float32),
        scratch_types=[
            pltpu.VMEM((_B_PER_W,), jnp.int32),
            pltpu.VMEM((_B_PER_W, ROW), jnp.float32),
            pltpu.SemaphoreType.DMA,
        ],
    )
    def sel(table_hbm, idx_hbm, out_hbm, idx_v, rows_v, sem):
        wid = lax.axis_index("s") * 2 + lax.axis_index("c")

        @pl.when(wid < _N_SEL_W)
        def _():
            base = wid * _B_PER_W
            pltpu.sync_copy(idx_hbm.at[pl.ds(base, _B_PER_W)], idx_v)
            pltpu.async_copy(table_hbm.at[idx_v], rows_v, sem).wait()
            pltpu.sync_copy(rows_v, out_hbm.at[pl.ds(base, _B_PER_W)])

    return sel


_SEL_CACHE = []


def _select(table, idx):
    if not _SEL_CACHE:
        _SEL_CACHE.append(_make_select())
    return _SEL_CACHE[0](table, idx)


def kernel(x, params):
    pT = params['enc_T']
    pS = params['enc_S']
    rT = pT['layers'][0]['m1']['W_dt'].shape[1]
    rS = pS['layers'][0]['m1']['W_dt'].shape[1]

    encT = functools.partial(_encoder, ep=pT, L=T, D=J3, r=rT, Cn=1, Q=T)
    encS = functools.partial(_encoder, ep=pS, L=J3, D=T, r=rS, Cn=1, Q=J3)

    u = encT(x.transpose(2, 0, 1))
    v = encS(x.transpose(1, 0, 2))

    i1, i2 = _gate(x, params['gate_w'], params['gate_b'])

    u_rows = u.transpose(1, 2, 0).reshape(B, ROW)
    v_rows = v.transpose(1, 0, 2).reshape(B, ROW)
    y1 = _select(jnp.concatenate([u_rows, v_rows], axis=0), i1)
    y1 = y1.reshape(B, J3, T)

    a = encS(y1.transpose(1, 0, 2))
    c = encT(y1.transpose(2, 0, 1))
    a_rows = a.transpose(1, 0, 2).reshape(B, ROW)
    c_rows = c.transpose(1, 2, 0).reshape(B, ROW)
    out = _select(jnp.concatenate([a_rows, c_rows], axis=0), i2)
    return out.reshape(B, J3, T)
